# baseline (speedup 1.0000x reference)
import functools

import jax
import jax.numpy as jnp
from jax import lax
from jax.experimental import pallas as pl
from jax.experimental.pallas import tpu as pltpu
from jax.experimental.pallas import tpu_sc as plsc

B, S, D = 4, 8192, 1024
TOTAL = B * S * D
POS_TOTAL = S * D
NW = 32
PER_W = TOTAL // NW
CHUNK = 32 * 1024
NCHUNK = PER_W // CHUNK
LANES = 16

_mesh = plsc.VectorSubcoreMesh(core_axis_name="c", subcore_axis_name="s")


@functools.partial(
    pl.kernel,
    mesh=_mesh,
    out_type=jax.ShapeDtypeStruct((TOTAL,), jnp.float32),
    scratch_types=[
        pltpu.VMEM((CHUNK,), jnp.float32),
        pltpu.VMEM((CHUNK,), jnp.float32),
        pltpu.SemaphoreType.DMA,
        pltpu.SemaphoreType.DMA,
    ],
)
def _sc_add(x_hbm, pos_hbm, out_hbm, xbuf, pbuf, sem_in, sem_out):
    wid = lax.axis_index("s") * 2 + lax.axis_index("c")
    x_off = wid * PER_W
    pos_off = (wid % 8) * PER_W

    def body(c, _):
        cx = x_off + c * CHUNK
        cp = pos_off + c * CHUNK
        pltpu.sync_copy(x_hbm.at[pl.ds(cx, CHUNK)], xbuf)
        pltpu.sync_copy(pos_hbm.at[pl.ds(cp, CHUNK)], pbuf)

        def add16(i, _):
            sl = pl.ds(i * LANES, LANES)
            xbuf[sl] = xbuf[sl] + pbuf[sl]
            return 0

        lax.fori_loop(0, CHUNK // LANES, add16, 0)
        pltpu.sync_copy(xbuf, out_hbm.at[pl.ds(cx, CHUNK)])
        return 0

    lax.fori_loop(0, NCHUNK, body, 0)


def kernel(x, pos_table):
    out = _sc_add(x.reshape(-1), pos_table.reshape(-1))
    return out.reshape(B, S, D)


# speedup vs baseline: 7.7726x; 7.7726x over previous
import jax
import jax.numpy as jnp
from jax.experimental import pallas as pl

B, S, D = 4, 8192, 1024
BLK_S = 1024


def _add_kernel(x_ref, pos_ref, o_ref):
    o_ref[...] = x_ref[...] + pos_ref[...]


def kernel(x, pos_table):
    grid = (S // BLK_S, B)
    return pl.pallas_call(
        _add_kernel,
        grid=grid,
        in_specs=[
            pl.BlockSpec((1, BLK_S, D), lambda s, b: (b, s, 0)),
            pl.BlockSpec((BLK_S, D), lambda s, b: (s, 0)),
        ],
        out_specs=pl.BlockSpec((1, BLK_S, D), lambda s, b: (b, s, 0)),
        out_shape=jax.ShapeDtypeStruct((B, S, D), x.dtype),
    )(x, pos_table)


# speedup vs baseline: 8.1000x; 1.0421x over previous
import jax
import jax.numpy as jnp
from jax.experimental import pallas as pl

B, S, D = 4, 8192, 1024
BLK_S = 2048


def _add_kernel(x_ref, pos_ref, o_ref):
    o_ref[...] = x_ref[...] + pos_ref[...]


def kernel(x, pos_table):
    grid = (S // BLK_S, B)
    return pl.pallas_call(
        _add_kernel,
        grid=grid,
        in_specs=[
            pl.BlockSpec((1, BLK_S, D), lambda s, b: (b, s, 0)),
            pl.BlockSpec((BLK_S, D), lambda s, b: (s, 0)),
        ],
        out_specs=pl.BlockSpec((1, BLK_S, D), lambda s, b: (b, s, 0)),
        out_shape=jax.ShapeDtypeStruct((B, S, D), x.dtype),
    )(x, pos_table)
